# baseline (speedup 1.0000x reference)
import functools

import jax
import jax.numpy as jnp
from jax import lax
from jax.experimental import pallas as pl
from jax.experimental.pallas import tpu as pltpu
from jax.experimental.pallas import tpu_sc as plsc


def _logits_body(x_ref, wr_ref, lgt_ref, probs_ref):
    lg = jnp.dot(x_ref[...], wr_ref[...],
                 preferred_element_type=jnp.float32)
    m1 = jnp.max(lg, axis=1, keepdims=True)
    ex = jnp.exp(lg - m1)
    probs_ref[...] = ex / jnp.sum(ex, axis=1, keepdims=True)
    lgt_ref[...] = lg.T


def _sc_router_body(lgt_hbm, maskt_hbm, i1_hbm, i2_hbm,
                    lg_v, mask_v, i1_v, i2_v, *, n_experts, tok_per_w,
                    n_cores):
    E = n_experts
    TW = tok_per_w
    wid = lax.axis_index("s") * n_cores + lax.axis_index("c")
    base = wid * TW
    pltpu.sync_copy(lgt_hbm.at[:, pl.ds(base, TW)], lg_v)

    def chunk(i, carry):
        del carry
        sl = pl.ds(i * 16, 16)
        ls = [lg_v[e, sl] for e in range(E)]
        m1 = ls[0]
        for e in range(1, E):
            m1 = jnp.maximum(m1, ls[e])
        i1 = jnp.full((16,), 0, jnp.int32)
        for e in range(E - 1, -1, -1):
            i1 = jnp.where(ls[e] == m1, jnp.full((16,), e, jnp.int32), i1)
        ninf = jnp.full((16,), -jnp.inf, jnp.float32)
        l2s = [jnp.where(i1 == e, ninf, ls[e]) for e in range(E)]
        m2 = l2s[0]
        for e in range(1, E):
            m2 = jnp.maximum(m2, l2s[e])
        i2 = jnp.full((16,), 0, jnp.int32)
        for e in range(E - 1, -1, -1):
            i2 = jnp.where(l2s[e] == m2, jnp.full((16,), e, jnp.int32), i2)
        b = jnp.exp(m2 - m1)
        g1 = 1.0 / (1.0 + b)
        g2 = 1.0 - g1
        zero = jnp.full((16,), 0.0, jnp.float32)
        for e in range(E):
            mask_v[e, sl] = jnp.where(i1 == e, g1,
                                      jnp.where(i2 == e, g2, zero))
        i1_v[sl] = i1
        i2_v[sl] = i2
        return 0

    lax.fori_loop(0, TW // 16, chunk, 0, unroll=False)
    pltpu.sync_copy(mask_v, maskt_hbm.at[:, pl.ds(base, TW)])
    pltpu.sync_copy(i1_v, i1_hbm.at[pl.ds(base, TW)])
    pltpu.sync_copy(i2_v, i2_hbm.at[pl.ds(base, TW)])


def _expert_body(x_ref, w1_ref, b1_ref, w2_ref, b2_ref, maskt_ref,
                 probs_ref, idx_ref, final_ref, act_ref, loss_ref,
                 acc_ref, lacc_ref, *, n_experts, tile, grid_dims,
                 n_tokens):
    c = pl.program_id(0)
    e = pl.program_id(1)
    t = pl.program_id(2)
    nc, ne, nti = grid_dims
    x = x_ref[...]
    h = jnp.dot(x, w1_ref[0], preferred_element_type=jnp.float32)
    h = jnp.maximum(h + b1_ref[0], 0.0)
    out = jnp.dot(h, w2_ref[0], preferred_element_type=jnp.float32)
    out = out + b2_ref[0]
    act_ref[0, 0, :] = jnp.mean(out, axis=1)
    m = maskt_ref[0, 0, :]
    contrib = out * m[:, None]
    sl = pl.ds(t * tile, tile)

    @pl.when((c == 0) & (e == 0) & (t == 0))
    def _():
        lacc_ref[...] = jnp.zeros_like(lacc_ref)

    @pl.when(e == 0)
    def _():
        acc_ref[sl, :] = contrib
        iota = jax.lax.broadcasted_iota(jnp.int32, probs_ref.shape, 1)
        ind = ((iota == idx_ref[:, 0:1]) | (iota == idx_ref[:, 1:2]))
        lacc_ref[0, :] += jnp.sum(ind.astype(jnp.float32), axis=0)
        lacc_ref[1, :] += jnp.sum(probs_ref[...], axis=0)

    @pl.when(e > 0)
    def _():
        acc_ref[sl, :] += contrib

    @pl.when(e == n_experts - 1)
    def _():
        final_ref[...] = acc_ref[sl, :]

    @pl.when((c == nc - 1) & (e == ne - 1) & (t == nti - 1))
    def _():
        denom = jnp.float32(n_tokens) * jnp.float32(n_tokens)
        loss = (jnp.float32(n_experts)
                * jnp.sum(lacc_ref[0, :] * lacc_ref[1, :]) / denom)
        loss_ref[...] = loss.reshape(1, 1)


def kernel(x, Wr, W1, b1, W2, b2):
    B, S, D = x.shape
    E = Wr.shape[1]
    F = W1.shape[2]
    BS = B * S
    xf = x.reshape(BS, D)

    T1 = 1024 if BS % 1024 == 0 else BS
    nt1 = BS // T1
    lgt, probs = pl.pallas_call(
        _logits_body,
        grid=(nt1,),
        in_specs=[
            pl.BlockSpec((T1, D), lambda t: (t, 0)),
            pl.BlockSpec((D, E), lambda t: (0, 0)),
        ],
        out_specs=[
            pl.BlockSpec((E, T1), lambda t: (0, t)),
            pl.BlockSpec((T1, E), lambda t: (t, 0)),
        ],
        out_shape=[
            jax.ShapeDtypeStruct((E, BS), jnp.float32),
            jax.ShapeDtypeStruct((BS, E), jnp.float32),
        ],
    )(xf, Wr)

    info = plsc.get_sparse_core_info()
    n_workers = info.num_cores * info.num_subcores
    tw = BS // n_workers
    mesh = plsc.VectorSubcoreMesh(core_axis_name="c", subcore_axis_name="s")
    maskt, i1, i2 = pl.kernel(
        functools.partial(_sc_router_body, n_experts=E, tok_per_w=tw,
                          n_cores=info.num_cores),
        mesh=mesh,
        out_type=[
            jax.ShapeDtypeStruct((E, BS), jnp.float32),
            jax.ShapeDtypeStruct((BS,), jnp.int32),
            jax.ShapeDtypeStruct((BS,), jnp.int32),
        ],
        scratch_types=[
            pltpu.VMEM((E, tw), jnp.float32),
            pltpu.VMEM((E, tw), jnp.float32),
            pltpu.VMEM((tw,), jnp.int32),
            pltpu.VMEM((tw,), jnp.int32),
        ],
    )(lgt)
    idx = jnp.stack([i1, i2], axis=1)

    W1c = W1.astype(jnp.bfloat16)
    W2c = W2.astype(jnp.bfloat16)
    T = 512 if BS % 512 == 0 else BS
    nc = 2 if (BS // T) % 2 == 0 else 1
    nti = BS // (T * nc)
    chunk = T * nti
    final, act, loss = pl.pallas_call(
        functools.partial(_expert_body, n_experts=E, tile=T,
                          grid_dims=(nc, E, nti), n_tokens=BS),
        grid=(nc, E, nti),
        in_specs=[
            pl.BlockSpec((T, D), lambda c, e, t: (c * nti + t, 0)),
            pl.BlockSpec((1, D, F), lambda c, e, t: (e, 0, 0)),
            pl.BlockSpec((1, 1, F), lambda c, e, t: (e, 0, 0)),
            pl.BlockSpec((1, F, D), lambda c, e, t: (e, 0, 0)),
            pl.BlockSpec((1, 1, D), lambda c, e, t: (e, 0, 0)),
            pl.BlockSpec((1, 1, T), lambda c, e, t: (e, 0, c * nti + t)),
            pl.BlockSpec((T, E), lambda c, e, t: (c * nti + t, 0)),
            pl.BlockSpec((T, 2), lambda c, e, t: (c * nti + t, 0)),
        ],
        out_specs=[
            pl.BlockSpec((T, D), lambda c, e, t: (c * nti + t, 0)),
            pl.BlockSpec((1, 1, T), lambda c, e, t: (e, 0, c * nti + t)),
            pl.BlockSpec((1, 1), lambda c, e, t: (0, 0)),
        ],
        out_shape=[
            jax.ShapeDtypeStruct((BS, D), jnp.float32),
            jax.ShapeDtypeStruct((E, 1, BS), jnp.float32),
            jax.ShapeDtypeStruct((1, 1), jnp.float32),
        ],
        scratch_shapes=[pltpu.VMEM((chunk, D), jnp.float32),
                        pltpu.VMEM((2, E), jnp.float32)],
    )(xf, W1c, b1.reshape(E, 1, F), W2c, b2.reshape(E, 1, D),
      maskt.reshape(E, 1, BS), probs, idx)

    return (final.reshape(B, S, D),
            act.reshape(E, B, S),
            maskt.T.reshape(B, S, E),
            loss[0, 0],
            probs.reshape(B, S, E),
            idx.reshape(B, S, 2))


# speedup vs baseline: 1.0047x; 1.0047x over previous
import functools

import jax
import jax.numpy as jnp
from jax import lax
from jax.experimental import pallas as pl
from jax.experimental.pallas import tpu as pltpu
from jax.experimental.pallas import tpu_sc as plsc


def _logits_body(x_ref, wr_ref, lgt_ref, probs_ref, psum_ref, pacc_ref,
                 *, n_tiles):
    t = pl.program_id(0)
    lg = jnp.dot(x_ref[...], wr_ref[...],
                 preferred_element_type=jnp.float32)
    m1 = jnp.max(lg, axis=1, keepdims=True)
    ex = jnp.exp(lg - m1)
    probs = ex / jnp.sum(ex, axis=1, keepdims=True)
    probs_ref[...] = probs
    lgt_ref[...] = lg.T

    @pl.when(t == 0)
    def _():
        pacc_ref[...] = jnp.zeros_like(pacc_ref)

    pacc_ref[0, :] += jnp.sum(probs, axis=0)

    @pl.when(t == n_tiles - 1)
    def _():
        psum_ref[...] = pacc_ref[...]


def _sc_router_body(lgt_hbm, maskt_hbm, i1_hbm, i2_hbm, cnt_hbm,
                    lg_v, mask_v, i1_v, i2_v, cnt_v, *, n_experts,
                    tok_per_w, n_cores):
    E = n_experts
    TW = tok_per_w
    wid = lax.axis_index("s") * n_cores + lax.axis_index("c")
    base = wid * TW
    pltpu.sync_copy(lgt_hbm.at[:, pl.ds(base, TW)], lg_v)
    zero16 = jnp.full((16,), 0.0, jnp.float32)
    for e in range(E):
        cnt_v[pl.ds(e * 16, 16)] = zero16

    def chunk(i, carry):
        del carry
        sl = pl.ds(i * 16, 16)
        ls = [lg_v[e, sl] for e in range(E)]
        m1 = ls[0]
        for e in range(1, E):
            m1 = jnp.maximum(m1, ls[e])
        i1 = jnp.full((16,), 0, jnp.int32)
        for e in range(E - 1, -1, -1):
            i1 = jnp.where(ls[e] == m1, jnp.full((16,), e, jnp.int32), i1)
        ninf = jnp.full((16,), -jnp.inf, jnp.float32)
        l2s = [jnp.where(i1 == e, ninf, ls[e]) for e in range(E)]
        m2 = l2s[0]
        for e in range(1, E):
            m2 = jnp.maximum(m2, l2s[e])
        i2 = jnp.full((16,), 0, jnp.int32)
        for e in range(E - 1, -1, -1):
            i2 = jnp.where(l2s[e] == m2, jnp.full((16,), e, jnp.int32), i2)
        b = jnp.exp(m2 - m1)
        g1 = 1.0 / (1.0 + b)
        g2 = 1.0 - g1
        zero = jnp.full((16,), 0.0, jnp.float32)
        one = jnp.full((16,), 1.0, jnp.float32)
        for e in range(E):
            mask_v[e, sl] = jnp.where(i1 == e, g1,
                                      jnp.where(i2 == e, g2, zero))
            ce = pl.ds(e * 16, 16)
            cnt_v[ce] += (jnp.where(i1 == e, one, zero)
                          + jnp.where(i2 == e, one, zero))
        i1_v[sl] = i1
        i2_v[sl] = i2
        return 0

    lax.fori_loop(0, TW // 16, chunk, 0, unroll=False)
    pltpu.sync_copy(mask_v, maskt_hbm.at[:, pl.ds(base, TW)])
    pltpu.sync_copy(i1_v, i1_hbm.at[pl.ds(base, TW)])
    pltpu.sync_copy(i2_v, i2_hbm.at[pl.ds(base, TW)])
    pltpu.sync_copy(cnt_v, cnt_hbm.at[pl.ds(wid * E * 16, E * 16)])


def _expert_body(x_ref, w1_ref, b1_ref, w2_ref, b2_ref, maskt_ref,
                 cnt_ref, psum_ref, final_ref, act_ref, loss_ref,
                 acc_ref, *, n_experts, tile, grid_dims, n_tokens):
    c = pl.program_id(0)
    e = pl.program_id(1)
    t = pl.program_id(2)
    nc, ne, nti = grid_dims
    x = x_ref[...]
    h = jnp.dot(x, w1_ref[0], preferred_element_type=jnp.float32)
    h = jnp.maximum(h + b1_ref[0], 0.0)
    out = jnp.dot(h, w2_ref[0], preferred_element_type=jnp.float32)
    out = out + b2_ref[0]
    act_ref[0, 0, :] = jnp.mean(out, axis=1)
    m = maskt_ref[0, 0, :]
    contrib = out * m[:, None]
    sl = pl.ds(t * tile, tile)

    @pl.when(e == 0)
    def _():
        acc_ref[sl, :] = contrib

    @pl.when(e > 0)
    def _():
        acc_ref[sl, :] += contrib

    @pl.when(e == n_experts - 1)
    def _():
        final_ref[...] = acc_ref[sl, :]

    @pl.when((c == nc - 1) & (e == ne - 1) & (t == nti - 1))
    def _():
        denom = jnp.float32(n_tokens) * jnp.float32(n_tokens)
        cnt = jnp.sum(cnt_ref[...], axis=(0, 2))
        loss = (jnp.float32(n_experts)
                * jnp.sum(cnt * psum_ref[0, :]) / denom)
        loss_ref[...] = loss.reshape(1, 1)


def kernel(x, Wr, W1, b1, W2, b2):
    B, S, D = x.shape
    E = Wr.shape[1]
    F = W1.shape[2]
    BS = B * S
    xf = x.reshape(BS, D)

    T1 = 1024 if BS % 1024 == 0 else BS
    nt1 = BS // T1
    lgt, probs, psum = pl.pallas_call(
        functools.partial(_logits_body, n_tiles=nt1),
        grid=(nt1,),
        in_specs=[
            pl.BlockSpec((T1, D), lambda t: (t, 0)),
            pl.BlockSpec((D, E), lambda t: (0, 0)),
        ],
        out_specs=[
            pl.BlockSpec((E, T1), lambda t: (0, t)),
            pl.BlockSpec((T1, E), lambda t: (t, 0)),
            pl.BlockSpec((1, E), lambda t: (0, 0)),
        ],
        out_shape=[
            jax.ShapeDtypeStruct((E, BS), jnp.float32),
            jax.ShapeDtypeStruct((BS, E), jnp.float32),
            jax.ShapeDtypeStruct((1, E), jnp.float32),
        ],
        scratch_shapes=[pltpu.VMEM((1, E), jnp.float32)],
    )(xf, Wr)

    info = plsc.get_sparse_core_info()
    n_workers = info.num_cores * info.num_subcores
    tw = BS // n_workers
    mesh = plsc.VectorSubcoreMesh(core_axis_name="c", subcore_axis_name="s")
    maskt, i1, i2, cnt = pl.kernel(
        functools.partial(_sc_router_body, n_experts=E, tok_per_w=tw,
                          n_cores=info.num_cores),
        mesh=mesh,
        out_type=[
            jax.ShapeDtypeStruct((E, BS), jnp.float32),
            jax.ShapeDtypeStruct((BS,), jnp.int32),
            jax.ShapeDtypeStruct((BS,), jnp.int32),
            jax.ShapeDtypeStruct((n_workers * E * 16,), jnp.float32),
        ],
        scratch_types=[
            pltpu.VMEM((E, tw), jnp.float32),
            pltpu.VMEM((E, tw), jnp.float32),
            pltpu.VMEM((tw,), jnp.int32),
            pltpu.VMEM((tw,), jnp.int32),
            pltpu.VMEM((E * 16,), jnp.float32),
        ],
    )(lgt)
    cnt = cnt.reshape(n_workers, E, 16)
    idx = jnp.stack([i1, i2], axis=1)

    W1c = W1.astype(jnp.bfloat16)
    W2c = W2.astype(jnp.bfloat16)
    T = 512 if BS % 512 == 0 else BS
    nc = 2 if (BS // T) % 2 == 0 else 1
    nti = BS // (T * nc)
    chunk = T * nti
    final, act, loss = pl.pallas_call(
        functools.partial(_expert_body, n_experts=E, tile=T,
                          grid_dims=(nc, E, nti), n_tokens=BS),
        grid=(nc, E, nti),
        in_specs=[
            pl.BlockSpec((T, D), lambda c, e, t: (c * nti + t, 0)),
            pl.BlockSpec((1, D, F), lambda c, e, t: (e, 0, 0)),
            pl.BlockSpec((1, 1, F), lambda c, e, t: (e, 0, 0)),
            pl.BlockSpec((1, F, D), lambda c, e, t: (e, 0, 0)),
            pl.BlockSpec((1, 1, D), lambda c, e, t: (e, 0, 0)),
            pl.BlockSpec((1, 1, T), lambda c, e, t: (e, 0, c * nti + t)),
            pl.BlockSpec(cnt.shape, lambda c, e, t: (0, 0, 0)),
            pl.BlockSpec((1, E), lambda c, e, t: (0, 0)),
        ],
        out_specs=[
            pl.BlockSpec((T, D), lambda c, e, t: (c * nti + t, 0)),
            pl.BlockSpec((1, 1, T), lambda c, e, t: (e, 0, c * nti + t)),
            pl.BlockSpec((1, 1), lambda c, e, t: (0, 0)),
        ],
        out_shape=[
            jax.ShapeDtypeStruct((BS, D), jnp.float32),
            jax.ShapeDtypeStruct((E, 1, BS), jnp.float32),
            jax.ShapeDtypeStruct((1, 1), jnp.float32),
        ],
        scratch_shapes=[pltpu.VMEM((chunk, D), jnp.float32)],
    )(xf, W1c, b1.reshape(E, 1, F), W2c, b2.reshape(E, 1, D),
      maskt.reshape(E, 1, BS), cnt, psum)

    return (final.reshape(B, S, D),
            act.reshape(E, B, S),
            maskt.T.reshape(B, S, E),
            loss[0, 0],
            probs.reshape(B, S, E),
            idx.reshape(B, S, 2))


# speedup vs baseline: 1.0056x; 1.0009x over previous
import functools

import jax
import jax.numpy as jnp
from jax import lax
from jax.experimental import pallas as pl
from jax.experimental.pallas import tpu as pltpu
from jax.experimental.pallas import tpu_sc as plsc


def _logits_body(x_ref, wr_ref, lgt_ref, probs_ref, psum_ref, pacc_ref,
                 *, n_tiles):
    t = pl.program_id(0)
    lg = jnp.dot(x_ref[...], wr_ref[...],
                 preferred_element_type=jnp.float32)
    m1 = jnp.max(lg, axis=1, keepdims=True)
    ex = jnp.exp(lg - m1)
    probs = ex / jnp.sum(ex, axis=1, keepdims=True)
    probs_ref[...] = probs
    lgt_ref[...] = lg.T

    @pl.when(t == 0)
    def _():
        pacc_ref[...] = jnp.zeros_like(pacc_ref)

    pacc_ref[0, :] += jnp.sum(probs, axis=0)

    @pl.when(t == n_tiles - 1)
    def _():
        psum_ref[...] = pacc_ref[...]


def _sc_router_body(lgt_hbm, maskt_hbm, i1_hbm, i2_hbm, cnt_hbm,
                    lg_v, mask_v, i1_v, i2_v, cnt_v, *, n_experts,
                    tok_per_w, n_cores):
    E = n_experts
    TW = tok_per_w
    wid = lax.axis_index("s") * n_cores + lax.axis_index("c")
    base = wid * TW
    pltpu.sync_copy(lgt_hbm.at[:, pl.ds(base, TW)], lg_v)
    zero16 = jnp.full((16,), 0.0, jnp.float32)
    for e in range(E):
        cnt_v[pl.ds(e * 16, 16)] = zero16

    def chunk(i, carry):
        del carry
        sl = pl.ds(i * 16, 16)
        ls = [lg_v[e, sl] for e in range(E)]
        m1 = ls[0]
        for e in range(1, E):
            m1 = jnp.maximum(m1, ls[e])
        i1 = jnp.full((16,), 0, jnp.int32)
        for e in range(E - 1, -1, -1):
            i1 = jnp.where(ls[e] == m1, jnp.full((16,), e, jnp.int32), i1)
        ninf = jnp.full((16,), -jnp.inf, jnp.float32)
        l2s = [jnp.where(i1 == e, ninf, ls[e]) for e in range(E)]
        m2 = l2s[0]
        for e in range(1, E):
            m2 = jnp.maximum(m2, l2s[e])
        i2 = jnp.full((16,), 0, jnp.int32)
        for e in range(E - 1, -1, -1):
            i2 = jnp.where(l2s[e] == m2, jnp.full((16,), e, jnp.int32), i2)
        b = jnp.exp(m2 - m1)
        g1 = 1.0 / (1.0 + b)
        g2 = 1.0 - g1
        zero = jnp.full((16,), 0.0, jnp.float32)
        one = jnp.full((16,), 1.0, jnp.float32)
        for e in range(E):
            mask_v[e, sl] = jnp.where(i1 == e, g1,
                                      jnp.where(i2 == e, g2, zero))
            ce = pl.ds(e * 16, 16)
            cnt_v[ce] += (jnp.where(i1 == e, one, zero)
                          + jnp.where(i2 == e, one, zero))
        i1_v[sl] = i1
        i2_v[sl] = i2
        return 0

    lax.fori_loop(0, TW // 16, chunk, 0, unroll=False)
    pltpu.sync_copy(mask_v, maskt_hbm.at[:, pl.ds(base, TW)])
    pltpu.sync_copy(i1_v, i1_hbm.at[pl.ds(base, TW)])
    pltpu.sync_copy(i2_v, i2_hbm.at[pl.ds(base, TW)])
    pltpu.sync_copy(cnt_v, cnt_hbm.at[pl.ds(wid * E * 16, E * 16)])


def _expert_body(x_ref, w1_ref, b1_ref, w2_ref, b2_ref, maskt_ref,
                 cnt_ref, psum_ref, final_ref, act_ref, loss_ref,
                 acc_ref, *, n_experts, tile, grid_dims, n_tokens):
    c = pl.program_id(0)
    e = pl.program_id(1)
    t = pl.program_id(2)
    nc, ne, nti = grid_dims
    x = x_ref[...]
    h = jnp.dot(x, w1_ref[0], preferred_element_type=jnp.float32)
    h = jnp.maximum(h + b1_ref[0], 0.0)
    out = jnp.dot(h, w2_ref[0], preferred_element_type=jnp.float32)
    out = out + b2_ref[0]
    act_ref[0, 0, :] = jnp.mean(out, axis=1)
    m = maskt_ref[0, 0, :]
    contrib = out * m[:, None]
    sl = pl.ds(t * tile, tile)

    @pl.when(e == 0)
    def _():
        acc_ref[sl, :] = contrib

    @pl.when(e > 0)
    def _():
        acc_ref[sl, :] += contrib

    @pl.when(e == n_experts - 1)
    def _():
        final_ref[...] = acc_ref[sl, :]

    @pl.when((c == nc - 1) & (e == ne - 1) & (t == nti - 1))
    def _():
        denom = jnp.float32(n_tokens) * jnp.float32(n_tokens)
        cnt = jnp.sum(cnt_ref[...], axis=(0, 2))
        loss = (jnp.float32(n_experts)
                * jnp.sum(cnt * psum_ref[0, :]) / denom)
        loss_ref[...] = loss.reshape(1, 1)


def kernel(x, Wr, W1, b1, W2, b2):
    B, S, D = x.shape
    E = Wr.shape[1]
    F = W1.shape[2]
    BS = B * S
    xf = x.reshape(BS, D)

    T1 = 1024 if BS % 1024 == 0 else BS
    nt1 = BS // T1
    lgt, probs, psum = pl.pallas_call(
        functools.partial(_logits_body, n_tiles=nt1),
        grid=(nt1,),
        in_specs=[
            pl.BlockSpec((T1, D), lambda t: (t, 0)),
            pl.BlockSpec((D, E), lambda t: (0, 0)),
        ],
        out_specs=[
            pl.BlockSpec((E, T1), lambda t: (0, t)),
            pl.BlockSpec((T1, E), lambda t: (t, 0)),
            pl.BlockSpec((1, E), lambda t: (0, 0)),
        ],
        out_shape=[
            jax.ShapeDtypeStruct((E, BS), jnp.float32),
            jax.ShapeDtypeStruct((BS, E), jnp.float32),
            jax.ShapeDtypeStruct((1, E), jnp.float32),
        ],
        scratch_shapes=[pltpu.VMEM((1, E), jnp.float32)],
    )(xf, Wr)

    info = plsc.get_sparse_core_info()
    n_workers = info.num_cores * info.num_subcores
    tw = BS // n_workers
    mesh = plsc.VectorSubcoreMesh(core_axis_name="c", subcore_axis_name="s")
    maskt, i1, i2, cnt = pl.kernel(
        functools.partial(_sc_router_body, n_experts=E, tok_per_w=tw,
                          n_cores=info.num_cores),
        mesh=mesh,
        out_type=[
            jax.ShapeDtypeStruct((E, BS), jnp.float32),
            jax.ShapeDtypeStruct((BS,), jnp.int32),
            jax.ShapeDtypeStruct((BS,), jnp.int32),
            jax.ShapeDtypeStruct((n_workers * E * 16,), jnp.float32),
        ],
        scratch_types=[
            pltpu.VMEM((E, tw), jnp.float32),
            pltpu.VMEM((E, tw), jnp.float32),
            pltpu.VMEM((tw,), jnp.int32),
            pltpu.VMEM((tw,), jnp.int32),
            pltpu.VMEM((E * 16,), jnp.float32),
        ],
    )(lgt)
    cnt = cnt.reshape(n_workers, E, 16)
    idx = jnp.stack([i1, i2], axis=1)

    W1c = W1.astype(jnp.bfloat16)
    W2c = W2.astype(jnp.bfloat16)
    T = 512 if BS % 512 == 0 else BS
    nc = 1
    nti = BS // (T * nc)
    chunk = T * nti
    final, act, loss = pl.pallas_call(
        functools.partial(_expert_body, n_experts=E, tile=T,
                          grid_dims=(nc, E, nti), n_tokens=BS),
        grid=(nc, E, nti),
        in_specs=[
            pl.BlockSpec((T, D), lambda c, e, t: (c * nti + t, 0)),
            pl.BlockSpec((1, D, F), lambda c, e, t: (e, 0, 0)),
            pl.BlockSpec((1, 1, F), lambda c, e, t: (e, 0, 0)),
            pl.BlockSpec((1, F, D), lambda c, e, t: (e, 0, 0)),
            pl.BlockSpec((1, 1, D), lambda c, e, t: (e, 0, 0)),
            pl.BlockSpec((1, 1, T), lambda c, e, t: (e, 0, c * nti + t)),
            pl.BlockSpec(cnt.shape, lambda c, e, t: (0, 0, 0)),
            pl.BlockSpec((1, E), lambda c, e, t: (0, 0)),
        ],
        out_specs=[
            pl.BlockSpec((T, D), lambda c, e, t: (c * nti + t, 0)),
            pl.BlockSpec((1, 1, T), lambda c, e, t: (e, 0, c * nti + t)),
            pl.BlockSpec((1, 1), lambda c, e, t: (0, 0)),
        ],
        out_shape=[
            jax.ShapeDtypeStruct((BS, D), jnp.float32),
            jax.ShapeDtypeStruct((E, 1, BS), jnp.float32),
            jax.ShapeDtypeStruct((1, 1), jnp.float32),
        ],
        scratch_shapes=[pltpu.VMEM((chunk, D), jnp.float32)],
    )(xf, W1c, b1.reshape(E, 1, F), W2c, b2.reshape(E, 1, D),
      maskt.reshape(E, 1, BS), cnt, psum)

    return (final.reshape(B, S, D),
            act.reshape(E, B, S),
            maskt.T.reshape(B, S, E),
            loss[0, 0],
            probs.reshape(B, S, E),
            idx.reshape(B, S, 2))
